# baseline (speedup 1.0000x reference)
import functools

import jax
import jax.numpy as jnp
from jax import lax
from jax.experimental import pallas as pl
from jax.experimental.pallas import tpu as pltpu
from jax.experimental.pallas import tpu_sc as plsc

L = 16
SEQ = 200
PAD_SEQ = 256
D = 128
NCH = D // L
NW2 = 8
INV_SEQ = 1.0 / SEQ


def _body(x0_hbm, x1_hbm, emb1_hbm, w1r_hbm, b1_hbm, emb2_hbm, out_hbm,
          idx_v, rows_v, part_v, pall_v, w1r_v, b1_v, x1_v, y2_v,
          cv, call_v, out_v, p_sh, c_sh, sem1, sem2):
    cid = lax.axis_index("c")
    sid = lax.axis_index("s")
    on0 = cid == 0

    @pl.when(on0)
    def _phase1():
        pltpu.sync_copy(x0_hbm.at[pl.ds(sid * L, L)], idx_v)
        pltpu.async_copy(emb1_hbm.at[idx_v], rows_v, sem1).wait()
        acc = [jnp.zeros((L,), jnp.float32) for _ in range(NCH)]
        for r in range(L):
            wgt = jnp.where(sid * L + r < SEQ, jnp.float32(1.0),
                            jnp.float32(0.0))
            for c in range(NCH):
                acc[c] = acc[c] + rows_v[r, pl.ds(c * L, L)] * wgt
        for c in range(NCH):
            part_v[pl.ds(c * L, L)] = acc[c]
        pltpu.sync_copy(part_v, p_sh.at[sid])

    plsc.subcore_barrier()

    @pl.when(on0 & (sid < NW2))
    def _phase2():
        pltpu.sync_copy(w1r_hbm.at[sid], w1r_v)
        pltpu.sync_copy(b1_hbm.at[pl.ds(sid * L, L)], b1_v)
        pltpu.sync_copy(x1_hbm, x1_v)
        pltpu.async_copy(emb2_hbm.at[x1_v], y2_v, sem2).wait()
        pltpu.sync_copy(p_sh, pall_v)
        s = [pall_v[0, pl.ds(c * L, L)] for c in range(NCH)]
        for t in range(1, L):
            for c in range(NCH):
                s[c] = s[c] + pall_v[t, pl.ds(c * L, L)]
        acc = jnp.zeros((L,), jnp.float32)
        for c in range(NCH):
            for l in range(L):
                acc = acc + s[c][l] * w1r_v[c * L + l, ...]
        y1 = acc * INV_SEQ + b1_v[...]
        y2 = y2_v[0, pl.ds(sid * L, L)]
        cv[...] = y1 * y2
        pltpu.sync_copy(cv, c_sh.at[sid])

    plsc.subcore_barrier()

    @pl.when(on0 & (sid == 0))
    def _final():
        pltpu.sync_copy(c_sh, call_v)
        tot = call_v[0, ...]
        for t in range(1, NW2):
            tot = tot + call_v[t, ...]
        tt = tot[0]
        for i in range(1, L):
            tt = tt + tot[i]
        tv = jnp.full((L,), tt, jnp.float32)
        out_v[...] = 1.0 / (1.0 + jnp.exp(-tv))
        pltpu.sync_copy(out_v, out_hbm)


@jax.jit
def _run(x0p, x1p, emb1, W1r, b1, emb2):
    mesh = plsc.VectorSubcoreMesh(core_axis_name="c", subcore_axis_name="s")
    call = functools.partial(
        pl.kernel,
        out_type=jax.ShapeDtypeStruct((L,), jnp.float32),
        mesh=mesh,
        scratch_types=[
            pltpu.VMEM((L,), jnp.int32),
            pltpu.VMEM((L, D), jnp.float32),
            pltpu.VMEM((D,), jnp.float32),
            pltpu.VMEM((L, D), jnp.float32),
            pltpu.VMEM((D, L), jnp.float32),
            pltpu.VMEM((L,), jnp.float32),
            pltpu.VMEM((8,), jnp.int32),
            pltpu.VMEM((8, D), jnp.float32),
            pltpu.VMEM((L,), jnp.float32),
            pltpu.VMEM((NW2, L), jnp.float32),
            pltpu.VMEM((L,), jnp.float32),
            pltpu.VMEM_SHARED((L, D), jnp.float32),
            pltpu.VMEM_SHARED((NW2, L), jnp.float32),
            pltpu.SemaphoreType.DMA,
            pltpu.SemaphoreType.DMA,
        ],
    )(_body)
    return call(x0p, x1p, emb1, W1r, b1, emb2)


def kernel(x0, x1, emb1, W1, b1, emb2):
    x0p = jnp.concatenate(
        [x0, jnp.zeros((PAD_SEQ - SEQ,), jnp.int32)])
    x1p = jnp.broadcast_to(x1, (8,))
    W1r = W1.T.reshape(D, NW2, L).transpose(1, 0, 2)
    out = _run(x0p, x1p, emb1, W1r, b1, emb2)
    return out[0]


# speedup vs baseline: 1.1582x; 1.1582x over previous
import functools

import jax
import jax.numpy as jnp
from jax import lax
from jax.experimental import pallas as pl
from jax.experimental.pallas import tpu as pltpu
from jax.experimental.pallas import tpu_sc as plsc

L = 16
SEQ = 200
PAD_SEQ = 256
D = 128
NCH = D // L
NW2 = 8
INV_SEQ = 1.0 / SEQ


def _body(x0_hbm, x1_hbm, emb1_hbm, w1r_hbm, b1_hbm, emb2_hbm, out_hbm,
          idx_v, rows_v, part_v, pall_v, w1r_v, b1_v, x1_v, y2_v,
          cv, call_v, out_v, p_sh, c_sh, sem1, sem2, semx):
    cid = lax.axis_index("c")
    sid = lax.axis_index("s")
    on0 = cid == 0
    isw = on0 & (sid < NW2)

    @pl.when(isw)
    def _prefetch():
        pltpu.async_copy(w1r_hbm.at[sid], w1r_v, sem2)
        pltpu.async_copy(b1_hbm.at[pl.ds(sid * L, L)], b1_v, sem2)
        pltpu.async_copy(x1_hbm, x1_v, semx)

    @pl.when(on0)
    def _fire_gather():
        pltpu.sync_copy(x0_hbm.at[pl.ds(sid * L, L)], idx_v)
        pltpu.async_copy(emb1_hbm.at[idx_v], rows_v, sem1)

    @pl.when(isw)
    def _fire_y2():
        pltpu.make_async_copy(x1_hbm, x1_v, semx).wait()
        pltpu.async_copy(emb2_hbm.at[x1_v], y2_v, sem2)

    @pl.when(on0)
    def _phase1():
        pltpu.make_async_copy(emb1_hbm.at[idx_v], rows_v, sem1).wait()
        acc = [jnp.zeros((L,), jnp.float32) for _ in range(NCH)]
        for r in range(L):
            wgt = jnp.where(sid * L + r < SEQ, jnp.float32(1.0),
                            jnp.float32(0.0))
            for c in range(NCH):
                acc[c] = acc[c] + rows_v[r, pl.ds(c * L, L)] * wgt
        for c in range(NCH):
            part_v[pl.ds(c * L, L)] = acc[c]
        pltpu.sync_copy(part_v, p_sh.at[sid])

    plsc.subcore_barrier()

    @pl.when(isw)
    def _phase2():
        pltpu.make_async_copy(w1r_hbm.at[sid], w1r_v, sem2).wait()
        pltpu.make_async_copy(b1_hbm.at[pl.ds(sid * L, L)], b1_v, sem2).wait()
        pltpu.make_async_copy(emb2_hbm.at[x1_v], y2_v, sem2).wait()
        pltpu.sync_copy(p_sh, pall_v)
        s = [pall_v[0, pl.ds(c * L, L)] for c in range(NCH)]
        for t in range(1, L):
            for c in range(NCH):
                s[c] = s[c] + pall_v[t, pl.ds(c * L, L)]
        acc = jnp.zeros((L,), jnp.float32)
        for c in range(NCH):
            for l in range(L):
                acc = acc + s[c][l] * w1r_v[c * L + l, ...]
        y1 = acc * INV_SEQ + b1_v[...]
        y2 = y2_v[0, pl.ds(sid * L, L)]
        cv[...] = y1 * y2
        pltpu.sync_copy(cv, c_sh.at[sid])

    plsc.subcore_barrier()

    @pl.when(on0 & (sid == 0))
    def _final():
        pltpu.sync_copy(c_sh, call_v)
        tot = call_v[0, ...]
        for t in range(1, NW2):
            tot = tot + call_v[t, ...]
        tt = tot[0]
        for i in range(1, L):
            tt = tt + tot[i]
        tv = jnp.full((L,), tt, jnp.float32)
        out_v[...] = 1.0 / (1.0 + jnp.exp(-tv))
        pltpu.sync_copy(out_v, out_hbm)


@jax.jit
def _run(x0p, x1p, emb1, W1r, b1, emb2):
    mesh = plsc.VectorSubcoreMesh(core_axis_name="c", subcore_axis_name="s")
    call = functools.partial(
        pl.kernel,
        out_type=jax.ShapeDtypeStruct((L,), jnp.float32),
        mesh=mesh,
        scratch_types=[
            pltpu.VMEM((L,), jnp.int32),
            pltpu.VMEM((L, D), jnp.float32),
            pltpu.VMEM((D,), jnp.float32),
            pltpu.VMEM((L, D), jnp.float32),
            pltpu.VMEM((D, L), jnp.float32),
            pltpu.VMEM((L,), jnp.float32),
            pltpu.VMEM((8,), jnp.int32),
            pltpu.VMEM((8, D), jnp.float32),
            pltpu.VMEM((L,), jnp.float32),
            pltpu.VMEM((NW2, L), jnp.float32),
            pltpu.VMEM((L,), jnp.float32),
            pltpu.VMEM_SHARED((L, D), jnp.float32),
            pltpu.VMEM_SHARED((NW2, L), jnp.float32),
            pltpu.SemaphoreType.DMA,
            pltpu.SemaphoreType.DMA,
            pltpu.SemaphoreType.DMA,
        ],
    )(_body)
    return call(x0p, x1p, emb1, W1r, b1, emb2)


def kernel(x0, x1, emb1, W1, b1, emb2):
    x0p = jnp.concatenate(
        [x0, jnp.zeros((PAD_SEQ - SEQ,), jnp.int32)])
    x1p = jnp.broadcast_to(x1, (8,))
    W1r = W1.T.reshape(D, NW2, L).transpose(1, 0, 2)
    out = _run(x0p, x1p, emb1, W1r, b1, emb2)
    return out[0]
